# baseline (speedup 1.0000x reference)
import jax
import jax.numpy as jnp
from jax.experimental import pallas as pl


def _copy_body(w_ref, o_ref):
    o_ref[...] = jnp.broadcast_to(w_ref[...][None], o_ref.shape)


def kernel(tokens, W_pos):
    batch, seq = tokens.shape
    d = W_pos.shape[1]
    blk = 2048
    bb = batch // 2
    out = pl.pallas_call(
        _copy_body,
        grid=(2, seq // blk),
        in_specs=[pl.BlockSpec((blk, d), lambda bh, j: (j, 0))],
        out_specs=pl.BlockSpec((bb, blk, d), lambda bh, j: (bh, j, 0)),
        out_shape=jax.ShapeDtypeStruct((batch, seq, d), W_pos.dtype),
    )(W_pos)
    return out


# speedup vs baseline: 1.1488x; 1.1488x over previous
import jax
import jax.numpy as jnp
from jax.experimental import pallas as pl


def _copy_body(w_ref, o_ref):
    o_ref[...] = jnp.broadcast_to(w_ref[...][None], o_ref.shape)


def kernel(tokens, W_pos):
    batch, seq = tokens.shape
    d = W_pos.shape[1]
    blk = 1024
    out = pl.pallas_call(
        _copy_body,
        grid=(seq // blk,),
        in_specs=[pl.BlockSpec((blk, d), lambda j: (j, 0))],
        out_specs=pl.BlockSpec((batch, blk, d), lambda j: (0, j, 0)),
        out_shape=jax.ShapeDtypeStruct((batch, seq, d), W_pos.dtype),
    )(W_pos)
    return out
